# Initial kernel scaffold; baseline (speedup 1.0000x reference)
import jax
import jax.numpy as jnp
from jax import lax
from jax.experimental import pallas as pl
from jax.experimental.pallas import tpu as pltpu
from jax.experimental.pallas import tpu_sc as plsc

N_NODES = 10000
N_EDGES = 160000
D_FEAT = 256
D_HALF = D_FEAT // 2

NUM_TILES = 16
CHUNK = 80
CHUNKS_TOTAL = N_EDGES // CHUNK
CHUNKS_PER_TILE = CHUNKS_TOTAL // NUM_TILES
ROWS_PER_TILE = N_NODES // NUM_TILES
ZROWS = 125


def _sc_body(x_hbm, src_hbm, dst_hbm, out_hbm,
             src_idx, dst_idx, rows, zbuf, acc, sem):
    c = lax.axis_index("c")
    s = lax.axis_index("s")

    @pl.loop(0, ZROWS)
    def _zero_rows(r):
        @pl.loop(0, D_HALF // 16)
        def _zero_lanes(j):
            zbuf[r, pl.ds(j * 16, 16)] = jnp.zeros((16,), jnp.float32)

    @pl.loop(0, ROWS_PER_TILE // ZROWS)
    def _clear(k):
        pltpu.sync_copy(zbuf, acc.at[pl.ds(s * ROWS_PER_TILE + k * ZROWS, ZROWS)])

    pltpu.sync_copy(src_hbm.at[pl.ds(s * CHUNKS_PER_TILE, CHUNKS_PER_TILE)], src_idx)
    pltpu.sync_copy(dst_hbm.at[pl.ds(s * CHUNKS_PER_TILE, CHUNKS_PER_TILE)], dst_idx)

    plsc.subcore_barrier()

    @pl.loop(0, CHUNKS_PER_TILE)
    def _edges(i):
        pltpu.async_copy(x_hbm.at[c].at[src_idx.at[i]], rows, sem).wait()
        pltpu.sync_copy(rows, acc.at[dst_idx.at[i]], add=True)

    plsc.subcore_barrier()

    pltpu.sync_copy(acc.at[pl.ds(s * ROWS_PER_TILE, ROWS_PER_TILE)],
                    out_hbm.at[c].at[pl.ds(s * ROWS_PER_TILE, ROWS_PER_TILE)])


@jax.jit
def _message_passing(x2, src_r, dst_r):
    mesh = plsc.VectorSubcoreMesh(core_axis_name="c", subcore_axis_name="s")
    run = pl.kernel(
        _sc_body,
        out_type=jax.ShapeDtypeStruct((2, N_NODES, D_HALF), jnp.float32),
        mesh=mesh,
        scratch_types=[
            pltpu.VMEM((CHUNKS_PER_TILE, CHUNK), jnp.int32),
            pltpu.VMEM((CHUNKS_PER_TILE, CHUNK), jnp.int32),
            pltpu.VMEM((CHUNK, D_HALF), jnp.float32),
            pltpu.VMEM((ZROWS, D_HALF), jnp.float32),
            pltpu.VMEM_SHARED((N_NODES, D_HALF), jnp.float32),
            pltpu.SemaphoreType.DMA,
        ],
    )
    return run(x2, src_r, dst_r)


def kernel(x, edge_index):
    x2 = jnp.moveaxis(x.reshape(N_NODES, 2, D_HALF), 1, 0)
    src_r = edge_index[0].reshape(CHUNKS_TOTAL, CHUNK)
    dst_r = edge_index[1].reshape(CHUNKS_TOTAL, CHUNK)
    out2 = _message_passing(x2, src_r, dst_r)
    return jnp.moveaxis(out2, 0, 1).reshape(N_NODES, D_FEAT)


# speedup vs baseline: 4.7567x; 4.7567x over previous
import jax
import jax.numpy as jnp
from jax import lax
from jax.experimental import pallas as pl
from jax.experimental.pallas import tpu as pltpu
from jax.experimental.pallas import tpu_sc as plsc

N_NODES = 10000
N_EDGES = 160000
D_FEAT = 256
D_HALF = D_FEAT // 2

NUM_TILES = 16
CHUNK = 80
CHUNKS_TOTAL = N_EDGES // CHUNK
CHUNKS_PER_TILE = CHUNKS_TOTAL // NUM_TILES
N_PAD = 10240
ROWS_PER_TILE = N_PAD // NUM_TILES
ZROWS = 128


def _sc_body(x_hbm, src_hbm, dst_hbm, out_hbm,
             src_idx, dst_idx, rows, zbuf, acc, sem):
    c = lax.axis_index("c")
    s = lax.axis_index("s")

    @pl.loop(0, ZROWS)
    def _zero_rows(r):
        @pl.loop(0, D_HALF // 16)
        def _zero_lanes(j):
            zbuf[r, pl.ds(j * 16, 16)] = jnp.zeros((16,), jnp.float32)

    @pl.loop(0, ROWS_PER_TILE // ZROWS)
    def _clear(k):
        pltpu.sync_copy(zbuf, acc.at[pl.ds(s * ROWS_PER_TILE + k * ZROWS, ZROWS)])

    pltpu.sync_copy(src_hbm.at[s], src_idx)
    pltpu.sync_copy(dst_hbm.at[s], dst_idx)

    plsc.subcore_barrier()

    @pl.loop(0, CHUNKS_PER_TILE)
    def _edges(i):
        pltpu.async_copy(x_hbm.at[c].at[src_idx.at[i]], rows, sem).wait()
        pltpu.sync_copy(rows, acc.at[dst_idx.at[i]], add=True)

    plsc.subcore_barrier()

    pltpu.sync_copy(acc.at[pl.ds(s * ROWS_PER_TILE, ROWS_PER_TILE)],
                    out_hbm.at[c].at[pl.ds(s * ROWS_PER_TILE, ROWS_PER_TILE)])


@jax.jit
def _message_passing(x2, src_r, dst_r):
    mesh = plsc.VectorSubcoreMesh(core_axis_name="c", subcore_axis_name="s")
    run = pl.kernel(
        _sc_body,
        out_type=jax.ShapeDtypeStruct((2, N_PAD, D_HALF), jnp.float32),
        mesh=mesh,
        scratch_types=[
            pltpu.VMEM((CHUNKS_PER_TILE, CHUNK), jnp.int32),
            pltpu.VMEM((CHUNKS_PER_TILE, CHUNK), jnp.int32),
            pltpu.VMEM((CHUNK, D_HALF), jnp.float32),
            pltpu.VMEM((ZROWS, D_HALF), jnp.float32),
            pltpu.VMEM_SHARED((N_PAD, D_HALF), jnp.float32),
            pltpu.SemaphoreType.DMA,
        ],
        compiler_params=pltpu.CompilerParams(use_tc_tiling_on_sc=False),
    )
    return run(x2, src_r, dst_r)


def kernel(x, edge_index):
    x2 = jnp.moveaxis(x.reshape(N_NODES, 2, D_HALF), 1, 0)
    src_r = edge_index[0].reshape(NUM_TILES, CHUNKS_PER_TILE, CHUNK)
    dst_r = edge_index[1].reshape(NUM_TILES, CHUNKS_PER_TILE, CHUNK)
    out2 = _message_passing(x2, src_r, dst_r)[:, :N_NODES, :]
    return jnp.moveaxis(out2, 0, 1).reshape(N_NODES, D_FEAT)


# speedup vs baseline: 5.8804x; 1.2362x over previous
import jax
import jax.numpy as jnp
from jax import lax
from jax.experimental import pallas as pl
from jax.experimental.pallas import tpu as pltpu
from jax.experimental.pallas import tpu_sc as plsc

N_NODES = 10000
N_EDGES = 160000
D_FEAT = 256
D_HALF = D_FEAT // 2

NUM_TILES = 16
CHUNK = 80
CHUNKS_TOTAL = N_EDGES // CHUNK
CHUNKS_PER_TILE = CHUNKS_TOTAL // NUM_TILES
N_PAD = 10240
ROWS_PER_TILE = N_PAD // NUM_TILES
ZROWS = 128
PAIRS = CHUNKS_PER_TILE // 2
TAIL = CHUNKS_PER_TILE - 2 * PAIRS


def _sc_body(x_hbm, src_hbm, dst_hbm, out_hbm,
             src_idx, dst_idx, rows0, rows1, acc,
             sem_g0, sem_g1):
    sem_s0, sem_s1 = sem_g0, sem_g1
    c = lax.axis_index("c")
    s = lax.axis_index("s")

    @pl.loop(0, CHUNK)
    def _zero_rows(r):
        @pl.loop(0, D_HALF // 16)
        def _zero_lanes(j):
            rows0[r, pl.ds(j * 16, 16)] = jnp.zeros((16,), jnp.float32)

    @pl.loop(0, ROWS_PER_TILE // CHUNK)
    def _clear(k):
        pltpu.sync_copy(rows0, acc.at[pl.ds(s * ROWS_PER_TILE + k * CHUNK, CHUNK)])

    pltpu.sync_copy(src_hbm.at[s], src_idx)
    pltpu.sync_copy(dst_hbm.at[s], dst_idx)

    plsc.subcore_barrier()

    def gather(j, rows, sem):
        return pltpu.async_copy(x_hbm.at[c].at[src_idx.at[j]], rows, sem)

    def wait_gather(j, rows, sem):
        pltpu.make_async_copy(x_hbm.at[c].at[src_idx.at[j]], rows, sem).wait()

    gather(0, rows0, sem_g0)
    gather(1, rows1, sem_g1)

    @pl.loop(0, PAIRS)
    def _edges(i):
        j0 = 2 * i
        wait_gather(j0, rows0, sem_g0)
        sc0 = pltpu.async_copy(rows0, acc.at[dst_idx.at[j0]], sem_s0, add=True)
        j1 = 2 * i + 1
        wait_gather(j1, rows1, sem_g1)
        sc1 = pltpu.async_copy(rows1, acc.at[dst_idx.at[j1]], sem_s1, add=True)
        jn0 = jnp.minimum(2 * i + 2, CHUNKS_PER_TILE - 1)
        jn1 = jnp.minimum(2 * i + 3, CHUNKS_PER_TILE - 1)
        sc0.wait()
        gather(jn0, rows0, sem_g0)
        sc1.wait()
        gather(jn1, rows1, sem_g1)

    last = CHUNKS_PER_TILE - 1
    wait_gather(last, rows0, sem_g0)
    pltpu.sync_copy(rows0, acc.at[dst_idx.at[last]], add=True)
    wait_gather(last, rows1, sem_g1)

    plsc.subcore_barrier()

    pltpu.sync_copy(acc.at[pl.ds(s * ROWS_PER_TILE, ROWS_PER_TILE)],
                    out_hbm.at[c].at[pl.ds(s * ROWS_PER_TILE, ROWS_PER_TILE)])


@jax.jit
def _message_passing(x2, src_r, dst_r):
    mesh = plsc.VectorSubcoreMesh(core_axis_name="c", subcore_axis_name="s")
    run = pl.kernel(
        _sc_body,
        out_type=jax.ShapeDtypeStruct((2, N_PAD, D_HALF), jnp.float32),
        mesh=mesh,
        scratch_types=[
            pltpu.VMEM((CHUNKS_PER_TILE, CHUNK), jnp.int32),
            pltpu.VMEM((CHUNKS_PER_TILE, CHUNK), jnp.int32),
            pltpu.VMEM((CHUNK, D_HALF), jnp.float32),
            pltpu.VMEM((CHUNK, D_HALF), jnp.float32),
            pltpu.VMEM_SHARED((N_PAD, D_HALF), jnp.float32),
            pltpu.SemaphoreType.DMA,
            pltpu.SemaphoreType.DMA,
        ],
        compiler_params=pltpu.CompilerParams(use_tc_tiling_on_sc=False),
    )
    return run(x2, src_r, dst_r)


def kernel(x, edge_index):
    x2 = jnp.moveaxis(x.reshape(N_NODES, 2, D_HALF), 1, 0)
    src_r = edge_index[0].reshape(NUM_TILES, CHUNKS_PER_TILE, CHUNK)
    dst_r = edge_index[1].reshape(NUM_TILES, CHUNKS_PER_TILE, CHUNK)
    out2 = _message_passing(x2, src_r, dst_r)[:, :N_NODES, :]
    return jnp.moveaxis(out2, 0, 1).reshape(N_NODES, D_FEAT)
